# baseline (speedup 1.0000x reference)
import functools

import jax
import jax.numpy as jnp
from jax import lax
from jax.experimental import pallas as pl
from jax.experimental.pallas import tpu as pltpu
from jax.experimental.pallas import tpu_sc as plsc

BATCH = 16384
EMBED = 32
LANES = 16
CHUNK = 128
PACK = 4
NUM_PACKED = 1000000 // PACK

_info = plsc.get_sparse_core_info()
_NC = _info.num_cores
_NS = _info.num_subcores
NW = _NC * _NS
BPW = BATCH // NW
NCHUNK = BPW // CHUNK
NGROUP = CHUNK // LANES


@functools.partial(
    pl.kernel,
    mesh=plsc.VectorSubcoreMesh(core_axis_name="c", subcore_axis_name="s"),
    out_type=jax.ShapeDtypeStruct((BATCH,), jnp.float32),
    compiler_params=pltpu.CompilerParams(needs_layout_passes=False),
    scratch_types=[
        pltpu.VMEM((BPW,), jnp.int32),
        pltpu.VMEM((BPW,), jnp.int32),
        pltpu.VMEM((BPW,), jnp.int32),
        pltpu.VMEM((BPW,), jnp.int32),
        pltpu.VMEM((CHUNK, PACK * EMBED), jnp.float32),
        pltpu.VMEM((CHUNK, PACK * EMBED), jnp.float32),
        pltpu.VMEM((BPW,), jnp.float32),
        pltpu.SemaphoreType.DMA,
        pltpu.SemaphoreType.DMA,
    ],
)
def _mf_kernel(user_hbm, item_hbm, ut_hbm, it_hbm, out_hbm,
               idx_u, idx_i, row_u, row_i, rows_u, rows_i, out_v,
               sem_u, sem_i):
    wid = lax.axis_index("s") * _NC + lax.axis_index("c")
    base = wid * BPW

    pltpu.sync_copy(user_hbm.at[pl.ds(base, BPW)], idx_u)
    pltpu.sync_copy(item_hbm.at[pl.ds(base, BPW)], idx_i)

    for k in range(BPW // LANES):
        s = pl.ds(k * LANES, LANES)
        row_u[s] = lax.shift_right_logical(idx_u[s], 2)
        row_i[s] = lax.shift_right_logical(idx_i[s], 2)

    lane_iota = lax.iota(jnp.int32, LANES)

    for c in range(NCHUNK):
        cu = pltpu.async_copy(
            ut_hbm.at[row_u.at[pl.ds(c * CHUNK, CHUNK)]], rows_u, sem_u)
        ci = pltpu.async_copy(
            it_hbm.at[row_i.at[pl.ds(c * CHUNK, CHUNK)]], rows_i, sem_i)
        cu.wait()
        ci.wait()
        for g in range(NGROUP):
            off = c * CHUNK + g * LANES
            rid = lane_iota + g * LANES
            qu = (idx_u[pl.ds(off, LANES)] & 3) * EMBED
            qi = (idx_i[pl.ds(off, LANES)] & 3) * EMBED
            acc = (plsc.load_gather(rows_u, [rid, qu])
                   * plsc.load_gather(rows_i, [rid, qi]))
            for d in range(1, EMBED):
                acc = acc + (plsc.load_gather(rows_u, [rid, qu + d])
                             * plsc.load_gather(rows_i, [rid, qi + d]))
            out_v[pl.ds(off, LANES)] = acc

    pltpu.sync_copy(out_v, out_hbm.at[pl.ds(base, BPW)])


def kernel(user, item, user_table, item_table):
    return _mf_kernel(user, item,
                      user_table.reshape(NUM_PACKED, PACK * EMBED),
                      item_table.reshape(NUM_PACKED, PACK * EMBED))
